# baseline (speedup 1.0000x reference)
import functools

import jax
import jax.numpy as jnp
from jax import lax
from jax.experimental import pallas as pl
from jax.experimental.pallas import tpu as pltpu
from jax.experimental.pallas import tpu_sc as plsc

T = 2048
D = 1024
F = 2048
E = 8
K = 2
S = T * K
BM = 128
SP = S + E * BM
NB = SP // BM
NBP = 48
NF = 2
FT = F // NF

_NC, _NS = 2, 16
NW = _NC * _NS
_GCH = 40
_GNC = SP // NW // _GCH


def _dispatch_gather_body(lg_hbm, x_hbm,
                          xs_hbm, gate_hbm, dest_hbm, b2e_hbm,
                          ids_v, w_v, rank_v, tok_v, gate_v, dest_v, b2e_v,
                          tok_sh, idx_v, rows0, rows1, sem0, sem1,
                          lg_v, e1_v, e2_v, w1_v, w2_v, ef_sh, wf_sh):
    cid = lax.axis_index("c")
    sid = lax.axis_index("s")

    TPW = T // _NS
    tb = sid * TPW
    pltpu.sync_copy(lg_hbm.at[pl.ds(tb, TPW)], lg_v)
    for g in range(TPW // 16):
        tk = lax.iota(jnp.int32, 16) + g * 16
        pe = [plsc.load_gather(lg_v, [tk, jnp.full((16,), e, jnp.int32)])
              for e in range(E)]
        m = pe[0]
        for e in range(1, E):
            m = jnp.maximum(m, pe[e])
        se = [jnp.exp(p - m) for p in pe]
        m1 = se[0]
        for e in range(1, E):
            m1 = jnp.maximum(m1, se[e])
        i1 = jnp.full((16,), E - 1, jnp.int32)
        for e in reversed(range(E)):
            i1 = jnp.where(se[e] == m1, e, i1)
        se2 = [jnp.where(i1 == e, jnp.float32(-1.0), se[e]) for e in range(E)]
        m2 = se2[0]
        for e in range(1, E):
            m2 = jnp.maximum(m2, se2[e])
        i2 = jnp.full((16,), E - 1, jnp.int32)
        for e in reversed(range(E)):
            i2 = jnp.where(se2[e] == m2, e, i2)
        ssum = m1 + m2
        e1_v[pl.ds(g * 16, 16)] = i1
        e2_v[pl.ds(g * 16, 16)] = i2
        w1_v[pl.ds(g * 16, 16)] = m1 / ssum
        w2_v[pl.ds(g * 16, 16)] = m2 / ssum
    pltpu.sync_copy(e1_v, ef_sh.at[pl.ds(tb, TPW)])
    pltpu.sync_copy(e2_v, ef_sh.at[pl.ds(T + tb, TPW)])
    pltpu.sync_copy(w1_v, wf_sh.at[pl.ds(tb, TPW)])
    pltpu.sync_copy(w2_v, wf_sh.at[pl.ds(T + tb, TPW)])
    plsc.subcore_barrier()

    @pl.when(sid == 0)
    def _():
        pltpu.sync_copy(ef_sh, ids_v)
        pltpu.sync_copy(wf_sh, w_v)

        def p1(i, base):
            v = ids_v[pl.ds(i * 16, 16)]
            rank = jnp.zeros((16,), jnp.int32)
            newbase = []
            for e in range(E):
                msk = v == e
                mi = msk.astype(jnp.int32)
                pre = plsc.cumsum(mi)
                rank = jnp.where(msk, base[e] + pre - 1, rank)
                newbase.append(base[e] + jnp.sum(mi))
            rank_v[pl.ds(i * 16, 16)] = rank
            return tuple(newbase)

        counts = lax.fori_loop(0, S // 16, p1, (jnp.int32(0),) * E)

        blk_end = []
        off_pad = []
        running = jnp.int32(0)
        for e in range(E):
            off_pad.append(running * BM)
            running = running + (counts[e] + (BM - 1)) // BM
            blk_end.append(running)

        def pz(j, _):
            z16i = jnp.zeros((16,), jnp.int32)
            z16f = jnp.zeros((16,), jnp.float32)
            tok_v[pl.ds(j * 16, 16)] = z16i
            gate_v[pl.ds(j * 16, 16)] = z16f
            return 0

        lax.fori_loop(0, SP // 16, pz, 0)

        def p2(i, _):
            v = ids_v[pl.ds(i * 16, 16)]
            r = rank_v[pl.ds(i * 16, 16)]
            off = jnp.zeros((16,), jnp.int32)
            for e in range(E):
                off = jnp.where(v == e, off_pad[e], off)
            d = off + r
            ivec = lax.iota(jnp.int32, 16) + i * 16
            tokid = ivec - jnp.where(ivec >= T, T, 0)
            plsc.store_scatter(tok_v, [d], tokid)
            plsc.store_scatter(gate_v, [d], w_v[pl.ds(i * 16, 16)])
            dest_v[pl.ds(i * 16, 16)] = d
            return 0

        lax.fori_loop(0, S // 16, p2, 0)

        for j in range(NBP // 16):
            sv = lax.iota(jnp.int32, 16) + j * 16
            acc = jnp.zeros((16,), jnp.int32)
            for e in range(E):
                acc = acc + (sv >= blk_end[e]).astype(jnp.int32)
            b2e_v[pl.ds(j * 16, 16)] = jnp.minimum(acc, E - 1)

        pltpu.sync_copy(tok_v, tok_sh)

        @pl.when(cid == 0)
        def _():
            pltpu.sync_copy(gate_v, gate_hbm)
            pltpu.sync_copy(dest_v, dest_hbm)
            pltpu.sync_copy(b2e_v, b2e_hbm)

    plsc.subcore_barrier()

    wid = sid * _NC + cid
    base = wid * (SP // NW)
    pltpu.sync_copy(tok_sh.at[pl.ds(base, SP // NW)], idx_v)
    rows = (rows0, rows1)
    sems = (sem0, sem1)
    cps = [None, None]
    cps[0] = pltpu.async_copy(x_hbm.at[idx_v.at[pl.ds(0, _GCH)]], rows0, sem0)
    for ch in range(_GNC):
        if ch + 1 < _GNC:
            nb = (ch + 1) % 2
            cps[nb] = pltpu.async_copy(
                x_hbm.at[idx_v.at[pl.ds((ch + 1) * _GCH, _GCH)]], rows[nb], sems[nb])
        cb = ch % 2
        cps[cb].wait()
        pltpu.sync_copy(rows[cb], xs_hbm.at[pl.ds(base + ch * _GCH, _GCH)])


def _dispatch_gather(router_logits, x):
    mesh = plsc.VectorSubcoreMesh(core_axis_name="c", subcore_axis_name="s")
    return pl.kernel(
        _dispatch_gather_body,
        out_type=[
            jax.ShapeDtypeStruct((SP, D // 2), jnp.int32),
            jax.ShapeDtypeStruct((SP,), jnp.float32),
            jax.ShapeDtypeStruct((S,), jnp.int32),
            jax.ShapeDtypeStruct((NBP,), jnp.int32),
        ],
        mesh=mesh,
        scratch_types=[
            pltpu.VMEM((S,), jnp.int32),
            pltpu.VMEM((S,), jnp.float32),
            pltpu.VMEM((S,), jnp.int32),
            pltpu.VMEM((SP,), jnp.int32),
            pltpu.VMEM((SP,), jnp.float32),
            pltpu.VMEM((S,), jnp.int32),
            pltpu.VMEM((NBP,), jnp.int32),
            pltpu.VMEM_SHARED((SP,), jnp.int32),
            pltpu.VMEM((SP // NW,), jnp.int32),
            pltpu.VMEM((_GCH, D // 2), jnp.int32),
            pltpu.VMEM((_GCH, D // 2), jnp.int32),
            pltpu.SemaphoreType.DMA,
            pltpu.SemaphoreType.DMA,
            pltpu.VMEM((T // _NS, E), jnp.float32),
            pltpu.VMEM((T // _NS,), jnp.int32),
            pltpu.VMEM((T // _NS,), jnp.int32),
            pltpu.VMEM((T // _NS,), jnp.float32),
            pltpu.VMEM((T // _NS,), jnp.float32),
            pltpu.VMEM_SHARED((S,), jnp.int32),
            pltpu.VMEM_SHARED((S,), jnp.float32),
        ],
        compiler_params=pltpu.CompilerParams(needs_layout_passes=False),
    )(router_logits, x)


def _ffn_body(b2e_ref, xs_ref, w11_ref, w12_ref, w2_ref, gate_ref, ys_ref):
    xb = xs_ref[...]
    g = jnp.dot(xb, w11_ref[0].astype(jnp.bfloat16),
                preferred_element_type=jnp.float32)
    v = jnp.dot(xb, w12_ref[0].astype(jnp.bfloat16),
                preferred_element_type=jnp.float32)
    h = g * jax.nn.sigmoid(g) * v
    y = jnp.dot(h, w2_ref[0], preferred_element_type=jnp.float32)
    ys_ref[...] = y * gate_ref[...]


def _ffn(b2e, xs, fc11, fc12, fc2, gate2d):
    grid_spec = pltpu.PrefetchScalarGridSpec(
        num_scalar_prefetch=1,
        grid=(NB,),
        in_specs=[
            pl.BlockSpec((BM, D), lambda b, b2e: (b, 0)),
            pl.BlockSpec((1, D, F), lambda b, b2e: (b2e[b], 0, 0)),
            pl.BlockSpec((1, D, F), lambda b, b2e: (b2e[b], 0, 0)),
            pl.BlockSpec((1, F, D), lambda b, b2e: (b2e[b], 0, 0)),
            pl.BlockSpec((BM, 1), lambda b, b2e: (b, 0)),
        ],
        out_specs=pl.BlockSpec((BM, D), lambda b, b2e: (b, 0)),
    )
    return pl.pallas_call(
        _ffn_body,
        grid_spec=grid_spec,
        out_shape=jax.ShapeDtypeStruct((SP, D), jnp.float32),
    )(b2e, xs, fc11, fc12, fc2, gate2d)


_CCH = 16
_CNC = T // NW // _CCH

def _combine_body(dest_hbm, ys_hbm, out_hbm, i0_v, i1_v,
                  r0a, r1a, r0b, r1b, s0a, s1a, s0b, s1b):
    cid = lax.axis_index("c")
    sid = lax.axis_index("s")
    wid = sid * _NC + cid
    base = wid * (T // NW)
    pltpu.sync_copy(dest_hbm.at[pl.ds(base, T // NW)], i0_v)
    pltpu.sync_copy(dest_hbm.at[pl.ds(T + base, T // NW)], i1_v)
    r0 = (r0a, r0b)
    r1 = (r1a, r1b)
    s0 = (s0a, s0b)
    s1 = (s1a, s1b)
    cps = [None, None]

    def issue(ch, b):
        c0 = pltpu.async_copy(ys_hbm.at[i0_v.at[pl.ds(ch * _CCH, _CCH)]], r0[b], s0[b])
        c1 = pltpu.async_copy(ys_hbm.at[i1_v.at[pl.ds(ch * _CCH, _CCH)]], r1[b], s1[b])
        return (c0, c1)

    cps[0] = issue(0, 0)
    for ch in range(_CNC):
        if ch + 1 < _CNC:
            cps[(ch + 1) % 2] = issue(ch + 1, (ch + 1) % 2)
        b = ch % 2
        cps[b][0].wait()
        cps[b][1].wait()

        def addrow(r, _):
            for j in range(D // 16):
                a = r0[b][r, pl.ds(j * 16, 16)]
                bb = r1[b][r, pl.ds(j * 16, 16)]
                r0[b][r, pl.ds(j * 16, 16)] = a + bb
            return 0

        lax.fori_loop(0, _CCH, addrow, 0)
        pltpu.sync_copy(r0[b], out_hbm.at[pl.ds(base + ch * _CCH, _CCH)])


def _combine(dest, ys):
    mesh = plsc.VectorSubcoreMesh(core_axis_name="c", subcore_axis_name="s")
    return pl.kernel(
        _combine_body,
        out_type=jax.ShapeDtypeStruct((T, D), jnp.float32),
        mesh=mesh,
        scratch_types=[
            pltpu.VMEM((T // NW,), jnp.int32),
            pltpu.VMEM((T // NW,), jnp.int32),
            pltpu.VMEM((_CCH, D), jnp.float32),
            pltpu.VMEM((_CCH, D), jnp.float32),
            pltpu.VMEM((_CCH, D), jnp.float32),
            pltpu.VMEM((_CCH, D), jnp.float32),
            pltpu.SemaphoreType.DMA,
            pltpu.SemaphoreType.DMA,
            pltpu.SemaphoreType.DMA,
            pltpu.SemaphoreType.DMA,
        ],
    )(dest, ys)


def kernel(x, router_logits, fc11, fc12, fc2):
    x2 = lax.bitcast_convert_type(
        x.astype(jnp.bfloat16).reshape(T, D // 2, 2), jnp.int32)
    xs2, gate, dest, b2e = _dispatch_gather(router_logits, x2)
    xs = lax.bitcast_convert_type(xs2, jnp.bfloat16).reshape(SP, D)
    ys = _ffn(b2e, xs, fc11, fc12, fc2, gate.reshape(SP, 1))
    out = _combine(dest, ys)
    return out


# speedup vs baseline: 1.5339x; 1.5339x over previous
import functools

import jax
import jax.numpy as jnp
from jax import lax
from jax.experimental import pallas as pl
from jax.experimental.pallas import tpu as pltpu
from jax.experimental.pallas import tpu_sc as plsc

T = 2048
D = 1024
F = 2048
E = 8
K = 2
S = T * K
BM = 128
SP = S + E * BM
NB = SP // BM
NBP = 48
NF = 2
FT = F // NF

_NC, _NS = 2, 16
NW = _NC * _NS
_GCH = 40
_GNC = SP // NW // _GCH


def _dispatch_gather_body(lg_hbm, x_hbm,
                          xs_hbm, gate_hbm, dest_hbm, b2e_hbm,
                          ids_v, w_v, rank_v, tok_v, gate_v, dest_v, b2e_v,
                          tok_sh, idx_v, rows0, rows1, sem0, sem1,
                          lg_v, e1_v, e2_v, w1_v, w2_v, ef_sh, wf_sh):
    cid = lax.axis_index("c")
    sid = lax.axis_index("s")

    TPW = T // _NS
    tb = sid * TPW
    pltpu.sync_copy(lg_hbm.at[pl.ds(tb, TPW)], lg_v)
    for g in range(TPW // 16):
        tk = lax.iota(jnp.int32, 16) + g * 16
        pe = [plsc.load_gather(lg_v, [tk, jnp.full((16,), e, jnp.int32)])
              for e in range(E)]
        m = pe[0]
        for e in range(1, E):
            m = jnp.maximum(m, pe[e])
        se = [jnp.exp(p - m) for p in pe]
        m1 = se[0]
        for e in range(1, E):
            m1 = jnp.maximum(m1, se[e])
        i1 = jnp.full((16,), E - 1, jnp.int32)
        for e in reversed(range(E)):
            i1 = jnp.where(se[e] == m1, e, i1)
        se2 = [jnp.where(i1 == e, jnp.float32(-1.0), se[e]) for e in range(E)]
        m2 = se2[0]
        for e in range(1, E):
            m2 = jnp.maximum(m2, se2[e])
        i2 = jnp.full((16,), E - 1, jnp.int32)
        for e in reversed(range(E)):
            i2 = jnp.where(se2[e] == m2, e, i2)
        ssum = m1 + m2
        e1_v[pl.ds(g * 16, 16)] = i1
        e2_v[pl.ds(g * 16, 16)] = i2
        w1_v[pl.ds(g * 16, 16)] = m1 / ssum
        w2_v[pl.ds(g * 16, 16)] = m2 / ssum
    pltpu.sync_copy(e1_v, ef_sh.at[pl.ds(tb, TPW)])
    pltpu.sync_copy(e2_v, ef_sh.at[pl.ds(T + tb, TPW)])
    pltpu.sync_copy(w1_v, wf_sh.at[pl.ds(tb, TPW)])
    pltpu.sync_copy(w2_v, wf_sh.at[pl.ds(T + tb, TPW)])
    plsc.subcore_barrier()

    @pl.when(sid == 0)
    def _():
        pltpu.sync_copy(ef_sh, ids_v)
        pltpu.sync_copy(wf_sh, w_v)

        def p1(i, base):
            v = ids_v[pl.ds(i * 16, 16)]
            rank = jnp.zeros((16,), jnp.int32)
            newbase = []
            for e in range(E):
                msk = v == e
                mi = msk.astype(jnp.int32)
                pre = plsc.cumsum(mi)
                rank = jnp.where(msk, base[e] + pre - 1, rank)
                newbase.append(base[e] + jnp.sum(mi))
            rank_v[pl.ds(i * 16, 16)] = rank
            return tuple(newbase)

        counts = lax.fori_loop(0, S // 16, p1, (jnp.int32(0),) * E)

        blk_end = []
        off_pad = []
        running = jnp.int32(0)
        for e in range(E):
            off_pad.append(running * BM)
            running = running + (counts[e] + (BM - 1)) // BM
            blk_end.append(running)

        def pz(j, _):
            z16i = jnp.zeros((16,), jnp.int32)
            z16f = jnp.zeros((16,), jnp.float32)
            tok_v[pl.ds(j * 16, 16)] = z16i
            gate_v[pl.ds(j * 16, 16)] = z16f
            return 0

        lax.fori_loop(0, SP // 16, pz, 0)

        def p2(i, _):
            v = ids_v[pl.ds(i * 16, 16)]
            r = rank_v[pl.ds(i * 16, 16)]
            off = jnp.zeros((16,), jnp.int32)
            for e in range(E):
                off = jnp.where(v == e, off_pad[e], off)
            d = off + r
            ivec = lax.iota(jnp.int32, 16) + i * 16
            tokid = ivec - jnp.where(ivec >= T, T, 0)
            plsc.store_scatter(tok_v, [d], tokid)
            plsc.store_scatter(gate_v, [d], w_v[pl.ds(i * 16, 16)])
            dest_v[pl.ds(i * 16, 16)] = d
            return 0

        lax.fori_loop(0, S // 16, p2, 0)

        for j in range(NBP // 16):
            sv = lax.iota(jnp.int32, 16) + j * 16
            acc = jnp.zeros((16,), jnp.int32)
            for e in range(E):
                acc = acc + (sv >= blk_end[e]).astype(jnp.int32)
            b2e_v[pl.ds(j * 16, 16)] = jnp.minimum(acc, E - 1)

        pltpu.sync_copy(tok_v, tok_sh)

        @pl.when(cid == 0)
        def _():
            pltpu.sync_copy(gate_v, gate_hbm)
            pltpu.sync_copy(dest_v, dest_hbm)
            pltpu.sync_copy(b2e_v, b2e_hbm)

    plsc.subcore_barrier()

    wid = sid * _NC + cid
    base = wid * (SP // NW)
    pltpu.sync_copy(tok_sh.at[pl.ds(base, SP // NW)], idx_v)
    rows = (rows0, rows1)
    sems = (sem0, sem1)
    cps = [None, None]
    cps[0] = pltpu.async_copy(x_hbm.at[idx_v.at[pl.ds(0, _GCH)]], rows0, sem0)
    for ch in range(_GNC):
        if ch + 1 < _GNC:
            nb = (ch + 1) % 2
            cps[nb] = pltpu.async_copy(
                x_hbm.at[idx_v.at[pl.ds((ch + 1) * _GCH, _GCH)]], rows[nb], sems[nb])
        cb = ch % 2
        cps[cb].wait()
        pltpu.sync_copy(rows[cb], xs_hbm.at[pl.ds(base + ch * _GCH, _GCH)])


def _dispatch_gather(router_logits, x):
    mesh = plsc.VectorSubcoreMesh(core_axis_name="c", subcore_axis_name="s")
    return pl.kernel(
        _dispatch_gather_body,
        out_type=[
            jax.ShapeDtypeStruct((SP, D), jnp.float32),
            jax.ShapeDtypeStruct((SP,), jnp.float32),
            jax.ShapeDtypeStruct((S,), jnp.int32),
            jax.ShapeDtypeStruct((NBP,), jnp.int32),
        ],
        mesh=mesh,
        scratch_types=[
            pltpu.VMEM((S,), jnp.int32),
            pltpu.VMEM((S,), jnp.float32),
            pltpu.VMEM((S,), jnp.int32),
            pltpu.VMEM((SP,), jnp.int32),
            pltpu.VMEM((SP,), jnp.float32),
            pltpu.VMEM((S,), jnp.int32),
            pltpu.VMEM((NBP,), jnp.int32),
            pltpu.VMEM_SHARED((SP,), jnp.int32),
            pltpu.VMEM((SP // NW,), jnp.int32),
            pltpu.VMEM((_GCH, D), jnp.float32),
            pltpu.VMEM((_GCH, D), jnp.float32),
            pltpu.SemaphoreType.DMA,
            pltpu.SemaphoreType.DMA,
            pltpu.VMEM((T // _NS, E), jnp.float32),
            pltpu.VMEM((T // _NS,), jnp.int32),
            pltpu.VMEM((T // _NS,), jnp.int32),
            pltpu.VMEM((T // _NS,), jnp.float32),
            pltpu.VMEM((T // _NS,), jnp.float32),
            pltpu.VMEM_SHARED((S,), jnp.int32),
            pltpu.VMEM_SHARED((S,), jnp.float32),
        ],
        compiler_params=pltpu.CompilerParams(needs_layout_passes=False),
    )(router_logits, x)


def _ffn_body(b2e_ref, xs_ref, w11_ref, w12_ref, w2_ref, gate_ref, ys_ref):
    xb = xs_ref[...]
    g = jnp.dot(xb, w11_ref[0], preferred_element_type=jnp.float32)
    v = jnp.dot(xb, w12_ref[0], preferred_element_type=jnp.float32)
    h = g * jax.nn.sigmoid(g) * v
    y = jnp.dot(h, w2_ref[0], preferred_element_type=jnp.float32)
    ys_ref[...] = y * gate_ref[...]


def _ffn(b2e, xs, fc11, fc12, fc2, gate2d):
    grid_spec = pltpu.PrefetchScalarGridSpec(
        num_scalar_prefetch=1,
        grid=(NB,),
        in_specs=[
            pl.BlockSpec((BM, D), lambda b, b2e: (b, 0)),
            pl.BlockSpec((1, D, F), lambda b, b2e: (b2e[b], 0, 0)),
            pl.BlockSpec((1, D, F), lambda b, b2e: (b2e[b], 0, 0)),
            pl.BlockSpec((1, F, D), lambda b, b2e: (b2e[b], 0, 0)),
            pl.BlockSpec((BM, 1), lambda b, b2e: (b, 0)),
        ],
        out_specs=pl.BlockSpec((BM, D), lambda b, b2e: (b, 0)),
    )
    return pl.pallas_call(
        _ffn_body,
        grid_spec=grid_spec,
        out_shape=jax.ShapeDtypeStruct((SP, D), jnp.float32),
    )(b2e, xs, fc11, fc12, fc2, gate2d)


_CCH = 16
_CNC = T // NW // _CCH

def _combine_body(dest_hbm, ys_hbm, out_hbm, i0_v, i1_v,
                  r0a, r1a, r0b, r1b, s0a, s1a, s0b, s1b):
    cid = lax.axis_index("c")
    sid = lax.axis_index("s")
    wid = sid * _NC + cid
    base = wid * (T // NW)
    pltpu.sync_copy(dest_hbm.at[pl.ds(base, T // NW)], i0_v)
    pltpu.sync_copy(dest_hbm.at[pl.ds(T + base, T // NW)], i1_v)
    r0 = (r0a, r0b)
    r1 = (r1a, r1b)
    s0 = (s0a, s0b)
    s1 = (s1a, s1b)
    cps = [None, None]

    def issue(ch, b):
        c0 = pltpu.async_copy(ys_hbm.at[i0_v.at[pl.ds(ch * _CCH, _CCH)]], r0[b], s0[b])
        c1 = pltpu.async_copy(ys_hbm.at[i1_v.at[pl.ds(ch * _CCH, _CCH)]], r1[b], s1[b])
        return (c0, c1)

    cps[0] = issue(0, 0)
    for ch in range(_CNC):
        if ch + 1 < _CNC:
            cps[(ch + 1) % 2] = issue(ch + 1, (ch + 1) % 2)
        b = ch % 2
        cps[b][0].wait()
        cps[b][1].wait()

        def addrow(r, _):
            for j in range(D // 16):
                a = r0[b][r, pl.ds(j * 16, 16)]
                bb = r1[b][r, pl.ds(j * 16, 16)]
                r0[b][r, pl.ds(j * 16, 16)] = a + bb
            return 0

        lax.fori_loop(0, _CCH, addrow, 0)
        pltpu.sync_copy(r0[b], out_hbm.at[pl.ds(base + ch * _CCH, _CCH)])


def _combine(dest, ys):
    mesh = plsc.VectorSubcoreMesh(core_axis_name="c", subcore_axis_name="s")
    return pl.kernel(
        _combine_body,
        out_type=jax.ShapeDtypeStruct((T, D), jnp.float32),
        mesh=mesh,
        scratch_types=[
            pltpu.VMEM((T // NW,), jnp.int32),
            pltpu.VMEM((T // NW,), jnp.int32),
            pltpu.VMEM((_CCH, D), jnp.float32),
            pltpu.VMEM((_CCH, D), jnp.float32),
            pltpu.VMEM((_CCH, D), jnp.float32),
            pltpu.VMEM((_CCH, D), jnp.float32),
            pltpu.SemaphoreType.DMA,
            pltpu.SemaphoreType.DMA,
            pltpu.SemaphoreType.DMA,
            pltpu.SemaphoreType.DMA,
        ],
    )(dest, ys)


def kernel(x, router_logits, fc11, fc12, fc2):
    xs, gate, dest, b2e = _dispatch_gather(router_logits, x)
    ys = _ffn(b2e, xs, fc11, fc12, fc2, gate.reshape(SP, 1))
    out = _combine(dest, ys)
    return out


# speedup vs baseline: 1.5562x; 1.0146x over previous
import functools

import jax
import jax.numpy as jnp
from jax import lax
from jax.experimental import pallas as pl
from jax.experimental.pallas import tpu as pltpu
from jax.experimental.pallas import tpu_sc as plsc

T = 2048
D = 1024
F = 2048
E = 8
K = 2
S = T * K
BM = 128
SP = S + E * BM
NB = SP // BM
NBP = 48
NF = 2
FT = F // NF

_NC, _NS = 2, 16
NW = _NC * _NS
_GCH = 40
_GNC = SP // NW // _GCH


def _dispatch_gather_body(lg_hbm, x_hbm,
                          xs_hbm, gate_hbm, dest_hbm, b2e_hbm,
                          ids_v, w_v, rank_v, tok_v, gate_v, dest_v, b2e_v,
                          tok_sh, idx_v, rows0, rows1, sem0, sem1,
                          lg_v, e1_v, e2_v, w1_v, w2_v, ef_sh, wf_sh):
    cid = lax.axis_index("c")
    sid = lax.axis_index("s")

    TPW = T // _NS
    tb = sid * TPW
    pltpu.sync_copy(lg_hbm.at[pl.ds(tb, TPW)], lg_v)
    for g in range(TPW // 16):
        tk = lax.iota(jnp.int32, 16) + g * 16
        pe = [plsc.load_gather(lg_v, [tk, jnp.full((16,), e, jnp.int32)])
              for e in range(E)]
        m = pe[0]
        for e in range(1, E):
            m = jnp.maximum(m, pe[e])
        se = [jnp.exp(p - m) for p in pe]
        m1 = se[0]
        for e in range(1, E):
            m1 = jnp.maximum(m1, se[e])
        i1 = jnp.full((16,), E - 1, jnp.int32)
        for e in reversed(range(E)):
            i1 = jnp.where(se[e] == m1, e, i1)
        se2 = [jnp.where(i1 == e, jnp.float32(-1.0), se[e]) for e in range(E)]
        m2 = se2[0]
        for e in range(1, E):
            m2 = jnp.maximum(m2, se2[e])
        i2 = jnp.full((16,), E - 1, jnp.int32)
        for e in reversed(range(E)):
            i2 = jnp.where(se2[e] == m2, e, i2)
        ssum = m1 + m2
        e1_v[pl.ds(g * 16, 16)] = i1
        e2_v[pl.ds(g * 16, 16)] = i2
        w1_v[pl.ds(g * 16, 16)] = m1 / ssum
        w2_v[pl.ds(g * 16, 16)] = m2 / ssum
    pltpu.sync_copy(e1_v, ef_sh.at[pl.ds(tb, TPW)])
    pltpu.sync_copy(e2_v, ef_sh.at[pl.ds(T + tb, TPW)])
    pltpu.sync_copy(w1_v, wf_sh.at[pl.ds(tb, TPW)])
    pltpu.sync_copy(w2_v, wf_sh.at[pl.ds(T + tb, TPW)])
    plsc.subcore_barrier()

    @pl.when(sid == 0)
    def _():
        pltpu.sync_copy(ef_sh, ids_v)
        pltpu.sync_copy(wf_sh, w_v)

        def p1(i, base):
            v = ids_v[pl.ds(i * 16, 16)]
            rank = jnp.zeros((16,), jnp.int32)
            newbase = []
            for e in range(E):
                msk = v == e
                mi = msk.astype(jnp.int32)
                pre = plsc.cumsum(mi)
                rank = jnp.where(msk, base[e] + pre - 1, rank)
                newbase.append(base[e] + jnp.sum(mi))
            rank_v[pl.ds(i * 16, 16)] = rank
            return tuple(newbase)

        counts = lax.fori_loop(0, S // 16, p1, (jnp.int32(0),) * E)

        blk_end = []
        off_pad = []
        running = jnp.int32(0)
        for e in range(E):
            off_pad.append(running * BM)
            running = running + (counts[e] + (BM - 1)) // BM
            blk_end.append(running)

        def pz(j, _):
            z16i = jnp.zeros((16,), jnp.int32)
            z16f = jnp.zeros((16,), jnp.float32)
            tok_v[pl.ds(j * 16, 16)] = z16i
            gate_v[pl.ds(j * 16, 16)] = z16f
            return 0

        lax.fori_loop(0, SP // 16, pz, 0)

        def p2(i, _):
            v = ids_v[pl.ds(i * 16, 16)]
            r = rank_v[pl.ds(i * 16, 16)]
            off = jnp.zeros((16,), jnp.int32)
            for e in range(E):
                off = jnp.where(v == e, off_pad[e], off)
            d = off + r
            ivec = lax.iota(jnp.int32, 16) + i * 16
            tokid = ivec - jnp.where(ivec >= T, T, 0)
            plsc.store_scatter(tok_v, [d], tokid)
            plsc.store_scatter(gate_v, [d], w_v[pl.ds(i * 16, 16)])
            dest_v[pl.ds(i * 16, 16)] = d
            return 0

        lax.fori_loop(0, S // 16, p2, 0)

        for j in range(NBP // 16):
            sv = lax.iota(jnp.int32, 16) + j * 16
            acc = jnp.zeros((16,), jnp.int32)
            for e in range(E):
                acc = acc + (sv >= blk_end[e]).astype(jnp.int32)
            vec = jnp.minimum(acc, E - 1)
            if j == NB // 16:
                vec = jnp.where(sv == NB, blk_end[E - 1], vec)
            b2e_v[pl.ds(j * 16, 16)] = vec

        pltpu.sync_copy(tok_v, tok_sh)

        @pl.when(cid == 0)
        def _():
            pltpu.sync_copy(gate_v, gate_hbm)
            pltpu.sync_copy(dest_v, dest_hbm)
            pltpu.sync_copy(b2e_v, b2e_hbm)

    plsc.subcore_barrier()

    wid = sid * _NC + cid
    base = wid * (SP // NW)
    pltpu.sync_copy(tok_sh.at[pl.ds(base, SP // NW)], idx_v)
    rows = (rows0, rows1)
    sems = (sem0, sem1)
    cps = [None, None]
    cps[0] = pltpu.async_copy(x_hbm.at[idx_v.at[pl.ds(0, _GCH)]], rows0, sem0)
    for ch in range(_GNC):
        if ch + 1 < _GNC:
            nb = (ch + 1) % 2
            cps[nb] = pltpu.async_copy(
                x_hbm.at[idx_v.at[pl.ds((ch + 1) * _GCH, _GCH)]], rows[nb], sems[nb])
        cb = ch % 2
        cps[cb].wait()
        pltpu.sync_copy(rows[cb], xs_hbm.at[pl.ds(base + ch * _GCH, _GCH)])


def _dispatch_gather(router_logits, x):
    mesh = plsc.VectorSubcoreMesh(core_axis_name="c", subcore_axis_name="s")
    return pl.kernel(
        _dispatch_gather_body,
        out_type=[
            jax.ShapeDtypeStruct((SP, D), jnp.float32),
            jax.ShapeDtypeStruct((SP,), jnp.float32),
            jax.ShapeDtypeStruct((S,), jnp.int32),
            jax.ShapeDtypeStruct((NBP,), jnp.int32),
        ],
        mesh=mesh,
        scratch_types=[
            pltpu.VMEM((S,), jnp.int32),
            pltpu.VMEM((S,), jnp.float32),
            pltpu.VMEM((S,), jnp.int32),
            pltpu.VMEM((SP,), jnp.int32),
            pltpu.VMEM((SP,), jnp.float32),
            pltpu.VMEM((S,), jnp.int32),
            pltpu.VMEM((NBP,), jnp.int32),
            pltpu.VMEM_SHARED((SP,), jnp.int32),
            pltpu.VMEM((SP // NW,), jnp.int32),
            pltpu.VMEM((_GCH, D), jnp.float32),
            pltpu.VMEM((_GCH, D), jnp.float32),
            pltpu.SemaphoreType.DMA,
            pltpu.SemaphoreType.DMA,
            pltpu.VMEM((T // _NS, E), jnp.float32),
            pltpu.VMEM((T // _NS,), jnp.int32),
            pltpu.VMEM((T // _NS,), jnp.int32),
            pltpu.VMEM((T // _NS,), jnp.float32),
            pltpu.VMEM((T // _NS,), jnp.float32),
            pltpu.VMEM_SHARED((S,), jnp.int32),
            pltpu.VMEM_SHARED((S,), jnp.float32),
        ],
        compiler_params=pltpu.CompilerParams(needs_layout_passes=False),
    )(router_logits, x)


def _ffn_body(b2e_ref, xs_ref, w11_ref, w12_ref, w2_ref, gate_ref, ys_ref):
    b = pl.program_id(0)

    @pl.when(b < b2e_ref[NB])
    def _():
        xb = xs_ref[...]
        g = jnp.dot(xb, w11_ref[0], preferred_element_type=jnp.float32)
        v = jnp.dot(xb, w12_ref[0], preferred_element_type=jnp.float32)
        h = g * jax.nn.sigmoid(g) * v
        y = jnp.dot(h, w2_ref[0], preferred_element_type=jnp.float32)
        ys_ref[...] = y * gate_ref[...]


def _ffn(b2e, xs, fc11, fc12, fc2, gate2d):
    grid_spec = pltpu.PrefetchScalarGridSpec(
        num_scalar_prefetch=1,
        grid=(NB,),
        in_specs=[
            pl.BlockSpec((BM, D), lambda b, b2e: (b, 0)),
            pl.BlockSpec((1, D, F), lambda b, b2e: (b2e[b], 0, 0)),
            pl.BlockSpec((1, D, F), lambda b, b2e: (b2e[b], 0, 0)),
            pl.BlockSpec((1, F, D), lambda b, b2e: (b2e[b], 0, 0)),
            pl.BlockSpec((BM, 1), lambda b, b2e: (b, 0)),
        ],
        out_specs=pl.BlockSpec((BM, D), lambda b, b2e: (b, 0)),
    )
    return pl.pallas_call(
        _ffn_body,
        grid_spec=grid_spec,
        out_shape=jax.ShapeDtypeStruct((SP, D), jnp.float32),
    )(b2e, xs, fc11, fc12, fc2, gate2d)


_CCH = 16
_CNC = T // NW // _CCH

def _combine_body(dest_hbm, ys_hbm, out_hbm, i0_v, i1_v,
                  r0a, r1a, r0b, r1b, s0a, s1a, s0b, s1b):
    cid = lax.axis_index("c")
    sid = lax.axis_index("s")
    wid = sid * _NC + cid
    base = wid * (T // NW)
    pltpu.sync_copy(dest_hbm.at[pl.ds(base, T // NW)], i0_v)
    pltpu.sync_copy(dest_hbm.at[pl.ds(T + base, T // NW)], i1_v)
    r0 = (r0a, r0b)
    r1 = (r1a, r1b)
    s0 = (s0a, s0b)
    s1 = (s1a, s1b)
    cps = [None, None]

    def issue(ch, b):
        c0 = pltpu.async_copy(ys_hbm.at[i0_v.at[pl.ds(ch * _CCH, _CCH)]], r0[b], s0[b])
        c1 = pltpu.async_copy(ys_hbm.at[i1_v.at[pl.ds(ch * _CCH, _CCH)]], r1[b], s1[b])
        return (c0, c1)

    cps[0] = issue(0, 0)
    for ch in range(_CNC):
        if ch + 1 < _CNC:
            cps[(ch + 1) % 2] = issue(ch + 1, (ch + 1) % 2)
        b = ch % 2
        cps[b][0].wait()
        cps[b][1].wait()

        def addrow(r, _):
            for j in range(D // 16):
                a = r0[b][r, pl.ds(j * 16, 16)]
                bb = r1[b][r, pl.ds(j * 16, 16)]
                r0[b][r, pl.ds(j * 16, 16)] = a + bb
            return 0

        lax.fori_loop(0, _CCH, addrow, 0)
        pltpu.sync_copy(r0[b], out_hbm.at[pl.ds(base + ch * _CCH, _CCH)])


def _combine(dest, ys):
    mesh = plsc.VectorSubcoreMesh(core_axis_name="c", subcore_axis_name="s")
    return pl.kernel(
        _combine_body,
        out_type=jax.ShapeDtypeStruct((T, D), jnp.float32),
        mesh=mesh,
        scratch_types=[
            pltpu.VMEM((T // NW,), jnp.int32),
            pltpu.VMEM((T // NW,), jnp.int32),
            pltpu.VMEM((_CCH, D), jnp.float32),
            pltpu.VMEM((_CCH, D), jnp.float32),
            pltpu.VMEM((_CCH, D), jnp.float32),
            pltpu.VMEM((_CCH, D), jnp.float32),
            pltpu.SemaphoreType.DMA,
            pltpu.SemaphoreType.DMA,
            pltpu.SemaphoreType.DMA,
            pltpu.SemaphoreType.DMA,
        ],
    )(dest, ys)


def kernel(x, router_logits, fc11, fc12, fc2):
    xs, gate, dest, b2e = _dispatch_gather(router_logits, x)
    ys = _ffn(b2e, xs, fc11, fc12, fc2, gate.reshape(SP, 1))
    out = _combine(dest, ys)
    return out
